# Initial kernel scaffold; baseline (speedup 1.0000x reference)
import jax
import jax.numpy as jnp
from jax.experimental import pallas as pl


def kernel(instances, instance_labels, memory):
    raise NotImplementedError("write your pallas kernel here")


# speedup vs baseline: 25.6065x; 25.6065x over previous
import jax
import jax.numpy as jnp
from jax.experimental import pallas as pl
from jax.experimental.pallas import tpu as pltpu

N_CLASSES = 1000
BANK = 10
DIM = 2048
BATCH = 4096

N_PAD = 1024
_MEAN_BLK = 8
_ROW_BLK = 512
_K_BLK = 256


def _bank_mean_kernel(mem_ref, out_ref):
    s = jnp.sum(mem_ref[...], axis=1) / BANK
    out_ref[...] = s - jnp.float32(1e-6)


def _score_kernel(x_ref, lbl_ref, mp_ref, idx_ref, hits_ref, acc_ref):
    j = pl.program_id(1)
    nk = pl.num_programs(1)
    mp = mp_ref[...]
    x = x_ref[...]
    norm2 = jnp.sum(mp * mp, axis=0).reshape(1, N_PAD)

    x1 = x.astype(jnp.bfloat16)
    xr = x - x1.astype(jnp.float32)
    x2 = xr.astype(jnp.bfloat16)
    x3 = (xr - x2.astype(jnp.float32)).astype(jnp.bfloat16)
    m1 = mp.astype(jnp.bfloat16)
    mr = mp - m1.astype(jnp.float32)
    m2 = mr.astype(jnp.bfloat16)
    m3 = (mr - m2.astype(jnp.float32)).astype(jnp.bfloat16)

    def bdot(a, b):
        return jax.lax.dot_general(
            a, b,
            dimension_numbers=(((1,), (0,)), ((), ())),
            preferred_element_type=jnp.float32,
        )

    cross = (bdot(x1, m3) + bdot(x2, m2) + bdot(x3, m1))
    cross += bdot(x1, m2) + bdot(x2, m1)
    cross += bdot(x1, m1)
    partial = norm2 - 2.0 * cross

    @pl.when(j == 0)
    def _():
        acc_ref[...] = jnp.zeros_like(acc_ref)

    acc_ref[...] += partial

    @pl.when(j == nk - 1)
    def _():
        cols = jax.lax.broadcasted_iota(jnp.int32, acc_ref.shape, 1)
        valid = cols < N_CLASSES
        s = jnp.where(valid, acc_ref[...], jnp.float32(jnp.inf))
        minval = jnp.min(s, axis=1, keepdims=True)
        fcols = cols.astype(jnp.float32)
        idxf = jnp.min(jnp.where(s == minval, fcols, jnp.float32(N_PAD)),
                       axis=1, keepdims=True)
        idx = idxf.astype(jnp.int32)
        idx_ref[...] = idx
        hits_ref[...] = jnp.sum((idx == lbl_ref[...]).astype(jnp.float32)
                                ).reshape(1, 1, 1)


def kernel(instances, instance_labels, memory):
    mp = pl.pallas_call(
        _bank_mean_kernel,
        grid=(N_CLASSES // _MEAN_BLK,),
        in_specs=[pl.BlockSpec((_MEAN_BLK, BANK, DIM), lambda i: (i, 0, 0))],
        out_specs=pl.BlockSpec((_MEAN_BLK, DIM), lambda i: (i, 0)),
        out_shape=jax.ShapeDtypeStruct((N_CLASSES, DIM), jnp.float32),
    )(memory)

    mp = jnp.pad(mp, ((0, N_PAD - N_CLASSES), (0, 0))).T
    labels = instance_labels.astype(jnp.int32)
    n_row_blocks = BATCH // _ROW_BLK
    idx, hits = pl.pallas_call(
        _score_kernel,
        grid=(n_row_blocks, DIM // _K_BLK),
        in_specs=[
            pl.BlockSpec((_ROW_BLK, _K_BLK), lambda i, j: (i, j)),
            pl.BlockSpec((_ROW_BLK, 1), lambda i, j: (i, 0)),
            pl.BlockSpec((_K_BLK, N_PAD), lambda i, j: (j, 0)),
        ],
        out_specs=[
            pl.BlockSpec((_ROW_BLK, 1), lambda i, j: (i, 0)),
            pl.BlockSpec((1, 1, 1), lambda i, j: (i, 0, 0)),
        ],
        out_shape=[
            jax.ShapeDtypeStruct((BATCH, 1), jnp.int32),
            jax.ShapeDtypeStruct((n_row_blocks, 1, 1), jnp.float32),
        ],
        scratch_shapes=[pltpu.VMEM((_ROW_BLK, N_PAD), jnp.float32)],
        compiler_params=pltpu.CompilerParams(
            dimension_semantics=("parallel", "arbitrary"),
        ),
    )(instances, labels, mp)

    cls_prob = idx.astype(instance_labels.dtype)
    rcnn_cls_acc = (jnp.sum(hits) / BATCH).astype(jnp.float32)
    return (cls_prob, rcnn_cls_acc)


# speedup vs baseline: 34.6452x; 1.3530x over previous
import jax
import jax.numpy as jnp
from jax.experimental import pallas as pl
from jax.experimental.pallas import tpu as pltpu

N_CLASSES = 1000
BANK = 10
DIM = 2048
BATCH = 4096

N_PAD = 1024
_K_BLK = 256
_ROW_BLK = 512


def _prep_kernel(mem_ref, m1_ref, m2_ref, m3_ref, norm2_ref):
    j = pl.program_id(0)
    mp = jnp.sum(mem_ref[...], axis=1) / BANK - jnp.float32(1e-6)
    mpt = jnp.concatenate(
        [mp.T, jnp.zeros((_K_BLK, N_PAD - N_CLASSES), jnp.float32)], axis=1)
    m1 = mpt.astype(jnp.bfloat16)
    r1 = mpt - m1.astype(jnp.float32)
    m2 = r1.astype(jnp.bfloat16)
    m3 = (r1 - m2.astype(jnp.float32)).astype(jnp.bfloat16)
    m1_ref[...] = m1
    m2_ref[...] = m2
    m3_ref[...] = m3
    part = jnp.sum(mpt * mpt, axis=0).reshape(1, N_PAD)

    @pl.when(j == 0)
    def _():
        norm2_ref[...] = jnp.zeros_like(norm2_ref)

    norm2_ref[...] += part


def _score_kernel(x_ref, lbl_ref, m1_ref, m2_ref, m3_ref, norm2_ref,
                  idx_ref, hits_ref):
    x = x_ref[...]
    x1 = x.astype(jnp.bfloat16)
    xr = x - x1.astype(jnp.float32)
    x2 = xr.astype(jnp.bfloat16)
    x3 = (xr - x2.astype(jnp.float32)).astype(jnp.bfloat16)

    def bdot(a, b_ref):
        return jax.lax.dot_general(
            a, b_ref[...],
            dimension_numbers=(((1,), (0,)), ((), ())),
            preferred_element_type=jnp.float32,
        )

    cross = bdot(x1, m3_ref) + bdot(x2, m2_ref) + bdot(x3, m1_ref)
    cross += bdot(x1, m2_ref) + bdot(x2, m1_ref)
    cross += bdot(x1, m1_ref)
    s = norm2_ref[...] - 2.0 * cross

    cols = jax.lax.broadcasted_iota(jnp.int32, s.shape, 1)
    s = jnp.where(cols < N_CLASSES, s, jnp.float32(jnp.inf))
    minval = jnp.min(s, axis=1, keepdims=True)
    fcols = cols.astype(jnp.float32)
    idxf = jnp.min(jnp.where(s == minval, fcols, jnp.float32(N_PAD)),
                   axis=1, keepdims=True)
    idx = idxf.astype(jnp.int32)
    idx_ref[...] = idx
    hits_ref[...] = jnp.sum((idx == lbl_ref[...]).astype(jnp.float32)
                            ).reshape(1, 1, 1)


def kernel(instances, instance_labels, memory):
    m1, m2, m3, norm2 = pl.pallas_call(
        _prep_kernel,
        grid=(DIM // _K_BLK,),
        in_specs=[pl.BlockSpec((N_CLASSES, BANK, _K_BLK),
                               lambda j: (0, 0, j))],
        out_specs=[
            pl.BlockSpec((_K_BLK, N_PAD), lambda j: (j, 0)),
            pl.BlockSpec((_K_BLK, N_PAD), lambda j: (j, 0)),
            pl.BlockSpec((_K_BLK, N_PAD), lambda j: (j, 0)),
            pl.BlockSpec((1, N_PAD), lambda j: (0, 0)),
        ],
        out_shape=[
            jax.ShapeDtypeStruct((DIM, N_PAD), jnp.bfloat16),
            jax.ShapeDtypeStruct((DIM, N_PAD), jnp.bfloat16),
            jax.ShapeDtypeStruct((DIM, N_PAD), jnp.bfloat16),
            jax.ShapeDtypeStruct((1, N_PAD), jnp.float32),
        ],
    )(memory)

    labels = instance_labels.astype(jnp.int32)
    n_row_blocks = BATCH // _ROW_BLK
    idx, hits = pl.pallas_call(
        _score_kernel,
        grid=(n_row_blocks,),
        in_specs=[
            pl.BlockSpec((_ROW_BLK, DIM), lambda i: (i, 0)),
            pl.BlockSpec((_ROW_BLK, 1), lambda i: (i, 0)),
            pl.BlockSpec((DIM, N_PAD), lambda i: (0, 0)),
            pl.BlockSpec((DIM, N_PAD), lambda i: (0, 0)),
            pl.BlockSpec((DIM, N_PAD), lambda i: (0, 0)),
            pl.BlockSpec((1, N_PAD), lambda i: (0, 0)),
        ],
        out_specs=[
            pl.BlockSpec((_ROW_BLK, 1), lambda i: (i, 0)),
            pl.BlockSpec((1, 1, 1), lambda i: (i, 0, 0)),
        ],
        out_shape=[
            jax.ShapeDtypeStruct((BATCH, 1), jnp.int32),
            jax.ShapeDtypeStruct((n_row_blocks, 1, 1), jnp.float32),
        ],
        compiler_params=pltpu.CompilerParams(
            dimension_semantics=("parallel",),
        ),
    )(instances, labels, m1, m2, m3, norm2)

    cls_prob = idx.astype(instance_labels.dtype)
    rcnn_cls_acc = (jnp.sum(hits) / BATCH).astype(jnp.float32)
    return (cls_prob, rcnn_cls_acc)
